# baseline (speedup 1.0000x reference)
import jax
import jax.numpy as jnp
from jax.experimental import pallas as pl
from jax.experimental.pallas import tpu as pltpu

_QT = 128
_GT = 32
_U = 16
_BIGQ = 1.0e9


def _body(pred_ref, anc_ref, tgt_ref, out_ref, tab_ref):
    i = pl.program_id(1)
    nsteps = pl.num_programs(1)
    G = tgt_ref.shape[1]
    tgt = tgt_ref[0]

    lane_f = jax.lax.broadcasted_iota(
        jnp.int32, (_GT, _QT), 1).astype(jnp.float32)

    @pl.when(i == 0)
    def _():
        tab_ref[:, 0:4] = jnp.full((4, 4, _GT, _QT), jnp.inf, jnp.float32)
        tab_ref[:, 4:8] = jnp.zeros((4, 4, _GT, _QT), jnp.float32)

    for mi, bx_ref in ((0, pred_ref), (4, anc_ref)):
        for h in range(G // _GT):
            grp = (mi // 4) * (G // _GT) + h
            th = tgt[h * _GT:(h + 1) * _GT]
            tcx = (th[:, 0:1] + th[:, 2:3]) / 2
            tcy = (th[:, 1:2] + th[:, 3:4]) / 2
            tw = th[:, 2:3] - th[:, 0:1]
            thh = th[:, 3:4] - th[:, 1:2]

            m1 = tab_ref[grp, 0]
            m2 = tab_ref[grp, 1]
            m3 = tab_ref[grp, 2]
            m4 = tab_ref[grp, 3]
            a1 = tab_ref[grp, 4]
            a2 = tab_ref[grp, 5]
            a3 = tab_ref[grp, 6]
            a4 = tab_ref[grp, 7]

            for k in range(_U):
                slab = bx_ref[0, k]
                x0 = slab[0:1, :]
                y0 = slab[1:2, :]
                x1 = slab[2:3, :]
                y1 = slab[3:4, :]
                cx = (x0 + x1) / 2
                cy = (y0 + y1) / 2
                w = x1 - x0
                hh = y1 - y0
                c = jnp.abs(cx - tcx)
                c = c + jnp.abs(cy - tcy)
                c = c + jnp.abs(w - tw)
                c = c + jnp.abs(hh - thh)
                tf = (i * _U + k).astype(jnp.float32)
                b1 = c < m1
                b2 = c < m2
                b3 = c < m3
                b4 = c < m4
                n1 = jnp.minimum(m1, c)
                n2 = jnp.minimum(m2, jnp.maximum(m1, c))
                n3 = jnp.minimum(m3, jnp.maximum(m2, c))
                n4 = jnp.minimum(m4, jnp.maximum(m3, c))
                a4 = jnp.where(b4, jnp.where(b3, a3, tf), a4)
                a3 = jnp.where(b3, jnp.where(b2, a2, tf), a3)
                a2 = jnp.where(b2, jnp.where(b1, a1, tf), a2)
                a1 = jnp.where(b1, tf, a1)
                m1, m2, m3, m4 = n1, n2, n3, n4

            tab_ref[grp, 0] = m1
            tab_ref[grp, 1] = m2
            tab_ref[grp, 2] = m3
            tab_ref[grp, 3] = m4
            tab_ref[grp, 4] = a1
            tab_ref[grp, 5] = a2
            tab_ref[grp, 6] = a3
            tab_ref[grp, 7] = a4

            @pl.when(i == nsteps - 1)
            def _(m1=m1, m2=m2, m3=m3, m4=m4,
                  a1=a1, a2=a2, a3=a3, a4=a4, h=h, mi=mi):
                cand_v = jnp.concatenate([m1, m2, m3, m4], axis=1)
                cand_q = jnp.concatenate(
                    [a1 * float(_QT) + lane_f, a2 * float(_QT) + lane_f,
                     a3 * float(_QT) + lane_f, a4 * float(_QT) + lane_f],
                    axis=1)
                for r in range(4):
                    gmin = jnp.min(cand_v, axis=1, keepdims=True)
                    qs = jnp.min(
                        jnp.where(cand_v == gmin, cand_q, _BIGQ),
                        axis=1, keepdims=True)
                    out_ref[0, h * _GT:(h + 1) * _GT,
                            mi + r:mi + r + 1] = qs.astype(jnp.int32)
                    if r < 3:
                        cand_v = jnp.where(cand_q == qs, jnp.inf, cand_v)


def kernel(pred_boxes, anchors, tgt_boxes):
    bs, Q, _ = pred_boxes.shape
    G = tgt_boxes.shape[1]
    K = 4
    tiles_per_step = _U
    nsteps = -(-Q // (_QT * _U))
    num_tiles = nsteps * _U
    Qp = num_tiles * _QT

    def prep(b):
        bp = jnp.pad(b, ((0, 0), (0, Qp - Q), (0, 0)),
                     constant_values=1e18)
        return bp.reshape(bs, num_tiles, _QT, 4).transpose(0, 1, 3, 2)

    pred_r = prep(pred_boxes)
    anc_r = prep(anchors)

    out = pl.pallas_call(
        _body,
        grid=(bs, nsteps),
        in_specs=[
            pl.BlockSpec((1, tiles_per_step, 4, _QT),
                         lambda b, i: (b, i, 0, 0)),
            pl.BlockSpec((1, tiles_per_step, 4, _QT),
                         lambda b, i: (b, i, 0, 0)),
            pl.BlockSpec((1, G, 4), lambda b, i: (b, 0, 0)),
        ],
        out_specs=pl.BlockSpec((1, G, 2 * K), lambda b, i: (b, 0, 0)),
        out_shape=jax.ShapeDtypeStruct((bs, G, 2 * K), jnp.int32),
        scratch_shapes=[pltpu.VMEM((4, 8, _GT, _QT), jnp.float32)],
    )(pred_r, anc_r, tgt_boxes)

    idx_i = out.reshape(bs, G * 2 * K).astype(jnp.int64)
    jrow = jnp.concatenate([jnp.arange(K), jnp.arange(K)])
    idx_j = jnp.tile(jrow, (bs, G)).astype(jnp.int64)
    return (idx_i, idx_j)


# speedup vs baseline: 1.1787x; 1.1787x over previous
import jax
import jax.numpy as jnp
from jax.experimental import pallas as pl

_QT = 128
_GT = 32
_U = 32
_BIGQ = 1.0e9


def _body(pred_ref, anc_ref, tgt_ref, out_ref):
    num_tiles = pred_ref.shape[1]
    G = tgt_ref.shape[1]
    tgt = tgt_ref[0]

    lane_f = jax.lax.broadcasted_iota(
        jnp.int32, (_GT, _QT), 1).astype(jnp.float32)

    for col0, bx_ref in ((0, pred_ref), (4, anc_ref)):
        for h in range(G // _GT):
            th = tgt[h * _GT:(h + 1) * _GT]
            tcx = (th[:, 0:1] + th[:, 2:3]) / 2
            tcy = (th[:, 1:2] + th[:, 3:4]) / 2
            tw = th[:, 2:3] - th[:, 0:1]
            thh = th[:, 3:4] - th[:, 1:2]

            inf = jnp.full((_GT, _QT), jnp.inf, jnp.float32)
            zero = jnp.zeros((_GT, _QT), jnp.float32)

            def step(t, carry, tcx=tcx, tcy=tcy, tw=tw, thh=thh,
                     bx_ref=bx_ref):
                m1, m2, m3, m4, a1, a2, a3, a4 = carry
                slab = bx_ref[0, t]
                x0 = slab[0:1, :]
                y0 = slab[1:2, :]
                x1 = slab[2:3, :]
                y1 = slab[3:4, :]
                cx = (x0 + x1) / 2
                cy = (y0 + y1) / 2
                w = x1 - x0
                hh = y1 - y0
                c = jnp.abs(cx - tcx)
                c = c + jnp.abs(cy - tcy)
                c = c + jnp.abs(w - tw)
                c = c + jnp.abs(hh - thh)
                tf = jnp.asarray(t, jnp.float32)
                b1 = c < m1
                b2 = c < m2
                b3 = c < m3
                b4 = c < m4
                n1 = jnp.minimum(m1, c)
                n2 = jnp.minimum(m2, jnp.maximum(m1, c))
                n3 = jnp.minimum(m3, jnp.maximum(m2, c))
                n4 = jnp.minimum(m4, jnp.maximum(m3, c))
                c4 = jnp.where(b4, jnp.where(b3, a3, tf), a4)
                c3 = jnp.where(b3, jnp.where(b2, a2, tf), a3)
                c2 = jnp.where(b2, jnp.where(b1, a1, tf), a2)
                c1 = jnp.where(b1, tf, a1)
                return (n1, n2, n3, n4, c1, c2, c3, c4)

            init = (inf, inf, inf, inf, zero, zero, zero, zero)
            full = num_tiles // _U

            def chunk(i, carry):
                t0 = i * _U
                for k in range(_U):
                    carry = step(t0 + k, carry)
                return carry

            carry = jax.lax.fori_loop(0, full, chunk, init)
            for t in range(full * _U, num_tiles):
                carry = step(t, carry)
            m1, m2, m3, m4, a1, a2, a3, a4 = carry

            cand_v = jnp.concatenate([m1, m2, m3, m4], axis=1)
            cand_q = jnp.concatenate(
                [a1 * float(_QT) + lane_f, a2 * float(_QT) + lane_f,
                 a3 * float(_QT) + lane_f, a4 * float(_QT) + lane_f],
                axis=1)
            for r in range(4):
                gmin = jnp.min(cand_v, axis=1, keepdims=True)
                qs = jnp.min(
                    jnp.where(cand_v == gmin, cand_q, _BIGQ),
                    axis=1, keepdims=True)
                out_ref[0, h * _GT:(h + 1) * _GT,
                        col0 + r:col0 + r + 1] = qs.astype(jnp.int32)
                if r < 3:
                    cand_v = jnp.where(cand_q == qs, jnp.inf, cand_v)


def kernel(pred_boxes, anchors, tgt_boxes):
    bs, Q, _ = pred_boxes.shape
    G = tgt_boxes.shape[1]
    K = 4
    num_tiles = (Q + _QT - 1) // _QT
    Qp = num_tiles * _QT

    def prep(b):
        bp = jnp.pad(b, ((0, 0), (0, Qp - Q), (0, 0)),
                     constant_values=1e18)
        return bp.reshape(bs, num_tiles, _QT, 4).transpose(0, 1, 3, 2)

    pred_r = prep(pred_boxes)
    anc_r = prep(anchors)

    out = pl.pallas_call(
        _body,
        grid=(bs,),
        in_specs=[
            pl.BlockSpec((1, num_tiles, 4, _QT), lambda b: (b, 0, 0, 0)),
            pl.BlockSpec((1, num_tiles, 4, _QT), lambda b: (b, 0, 0, 0)),
            pl.BlockSpec((1, G, 4), lambda b: (b, 0, 0)),
        ],
        out_specs=pl.BlockSpec((1, G, 2 * K), lambda b: (b, 0, 0)),
        out_shape=jax.ShapeDtypeStruct((bs, G, 2 * K), jnp.int32),
    )(pred_r, anc_r, tgt_boxes)

    idx_i = out.reshape(bs, G * 2 * K).astype(jnp.int64)
    jrow = jnp.concatenate([jnp.arange(K), jnp.arange(K)])
    idx_j = jnp.tile(jrow, (bs, G)).astype(jnp.int64)
    return (idx_i, idx_j)
